# Initial kernel scaffold; baseline (speedup 1.0000x reference)
import functools

import jax
import jax.numpy as jnp
from jax import lax
from jax.experimental import pallas as pl
from jax.experimental.pallas import tpu as pltpu
from jax.experimental.pallas import tpu_sc as plsc

EMB = 128
BATCH = 16384

_INFO = plsc.get_sparse_core_info()
NC = _INFO.num_cores
NS = _INFO.num_subcores
L = _INFO.num_lanes
NW = NC * NS
ROWS_PER_W = BATCH // NW
CHUNK = 128
NCHUNK = ROWS_PER_W // CHUNK


def _body(u_hbm, v_hbm, user_hbm, item_hbm, out_hbm,
          uidx_v, vidx_v, ue_v, ve_v, out_v, sem):
    wid = lax.axis_index("s") * NC + lax.axis_index("c")
    base = wid * ROWS_PER_W

    lanes = lax.iota(jnp.int32, L)

    for c in range(NCHUNK):
        pltpu.sync_copy(u_hbm.at[pl.ds(base + c * CHUNK, CHUNK)], uidx_v.at[c])
        pltpu.sync_copy(v_hbm.at[pl.ds(base + c * CHUNK, CHUNK)], vidx_v.at[c])
        cp_u = pltpu.make_async_copy(user_hbm.at[uidx_v.at[c]], ue_v, sem)
        cp_v = pltpu.make_async_copy(item_hbm.at[vidx_v.at[c]], ve_v, sem)
        cp_u.start()
        cp_v.start()
        cp_u.wait()
        cp_v.wait()

        for g in range(CHUNK // L):
            rows = g * L + lanes

            def col_body(j, acc):
                cols = jnp.full((L,), j, dtype=jnp.int32)
                cu = plsc.load_gather(ue_v, [rows, cols])
                cv = plsc.load_gather(ve_v, [rows, cols])
                return acc + cu * cv

            acc = lax.fori_loop(0, EMB, col_body, jnp.zeros((L,), jnp.float32))
            out_v[pl.ds(c * CHUNK + g * L, L)] = acc

    pltpu.sync_copy(out_v, out_hbm.at[pl.ds(base, ROWS_PER_W)])


@jax.jit
def kernel(u, v, user_emb, item_emb):
    mesh = plsc.VectorSubcoreMesh(core_axis_name="c", subcore_axis_name="s")
    run = pl.kernel(
        _body,
        mesh=mesh,
        out_type=jax.ShapeDtypeStruct((BATCH,), jnp.float32),
        scratch_types=[
            pltpu.VMEM((NCHUNK, CHUNK), jnp.int32),
            pltpu.VMEM((NCHUNK, CHUNK), jnp.int32),
            pltpu.VMEM((CHUNK, EMB), jnp.float32),
            pltpu.VMEM((CHUNK, EMB), jnp.float32),
            pltpu.VMEM((ROWS_PER_W,), jnp.float32),
            pltpu.SemaphoreType.DMA,
        ],
    )
    return run(u, v, user_emb, item_emb)


# speedup vs baseline: 1.2757x; 1.2757x over previous
import functools

import jax
import jax.numpy as jnp
from jax import lax
from jax.experimental import pallas as pl
from jax.experimental.pallas import tpu as pltpu
from jax.experimental.pallas import tpu_sc as plsc

EMB = 128
BATCH = 16384

_INFO = plsc.get_sparse_core_info()
NC = _INFO.num_cores
NS = _INFO.num_subcores
L = _INFO.num_lanes
NW = NC * NS
ROWS_PER_W = BATCH // NW
CHUNK = 128
NCHUNK = ROWS_PER_W // CHUNK


def _lane_shuffle(x, idx):
    dnums = lax.GatherDimensionNumbers(
        offset_dims=(), collapsed_slice_dims=(0,), start_index_map=(0,))
    return lax.gather(x, idx[:, None], dnums, (1,),
                      mode=lax.GatherScatterMode.PROMISE_IN_BOUNDS)


def _body(u_hbm, v_hbm, user_hbm, item_hbm, out_hbm,
          uidx_v, vidx_v, ue_v, ve_v, out_v, sem):
    wid = lax.axis_index("s") * NC + lax.axis_index("c")
    base = wid * ROWS_PER_W

    lanes = lax.iota(jnp.int32, L)

    for c in range(NCHUNK):
        pltpu.sync_copy(u_hbm.at[pl.ds(base + c * CHUNK, CHUNK)], uidx_v.at[c])
        pltpu.sync_copy(v_hbm.at[pl.ds(base + c * CHUNK, CHUNK)], vidx_v.at[c])
        cp_u = pltpu.make_async_copy(user_hbm.at[uidx_v.at[c]], ue_v, sem)
        cp_v = pltpu.make_async_copy(item_hbm.at[vidx_v.at[c]], ve_v, sem)
        cp_u.start()
        cp_v.start()
        cp_u.wait()
        cp_v.wait()

        def group_body(g, _):
            def row_body(r, vec):
                ridx = g * L + r
                urow = ue_v.at[ridx]
                vrow = ve_v.at[ridx]
                acc = urow[pl.ds(0, L)] * vrow[pl.ds(0, L)]
                for k in range(1, EMB // L):
                    acc = acc + urow[pl.ds(k * L, L)] * vrow[pl.ds(k * L, L)]
                for sh in (8, 4, 2, 1):
                    acc = acc + _lane_shuffle(acc, lanes ^ sh)
                return jnp.where(lanes == r, acc, vec)

            vec = lax.fori_loop(0, L, row_body, jnp.zeros((L,), jnp.float32))
            out_v[pl.ds(c * CHUNK + g * L, L)] = vec
            return 0

        lax.fori_loop(0, CHUNK // L, group_body, 0)

    pltpu.sync_copy(out_v, out_hbm.at[pl.ds(base, ROWS_PER_W)])


@jax.jit
def kernel(u, v, user_emb, item_emb):
    mesh = plsc.VectorSubcoreMesh(core_axis_name="c", subcore_axis_name="s")
    run = pl.kernel(
        _body,
        mesh=mesh,
        out_type=jax.ShapeDtypeStruct((BATCH,), jnp.float32),
        scratch_types=[
            pltpu.VMEM((NCHUNK, CHUNK), jnp.int32),
            pltpu.VMEM((NCHUNK, CHUNK), jnp.int32),
            pltpu.VMEM((CHUNK, EMB), jnp.float32),
            pltpu.VMEM((CHUNK, EMB), jnp.float32),
            pltpu.VMEM((ROWS_PER_W,), jnp.float32),
            pltpu.SemaphoreType.DMA,
        ],
    )
    return run(u, v, user_emb, item_emb)
